# Initial kernel scaffold; baseline (speedup 1.0000x reference)
import jax
import jax.numpy as jnp
from jax.experimental import pallas as pl


def kernel(x, edge_index, hyperedge_index, W_g1, b_g1, W_g2, b_g2, W_h1, b_h1, W_h2, b_h2, gate):
    raise NotImplementedError("write your pallas kernel here")


# speedup vs baseline: 14.4543x; 14.4543x over previous
import jax
import jax.numpy as jnp
from jax import lax
from jax.experimental import pallas as pl
from jax.experimental.pallas import tpu as pltpu
from jax.experimental.pallas import tpu_sc as plsc

f32 = jnp.float32

NN = 10000
NP = 10240
FD = 128
NE = 320000
K = 80
NSUB = 16
NCORE = 2
RPS = NP // NSUB
STG = 128
CW = 16

_MESH = plsc.VectorSubcoreMesh(
    core_axis_name="c", subcore_axis_name="s",
    num_cores=NCORE, num_subcores=NSUB)


def _stage_rows(src, dst, base, zb):
  for t in range(RPS // STG):
    pltpu.sync_copy(src.at[pl.ds(base + t * STG, STG)], zb)
    pltpu.sync_copy(zb, dst.at[pl.ds(base + t * STG, STG)])


def _fill_rows(zsrc, dst, base, zb):
  pltpu.sync_copy(zsrc, zb)
  for t in range(RPS // STG):
    pltpu.sync_copy(zb, dst.at[pl.ds(base + t * STG, STG)])


def _scatter_stream(nch, c0, table, src1, dst1, acc,
                    sb0, sb1, db0, db1, r0, r1, si0, si1, sg0, sg1):
  def ist(j, sb, db, si):
    eb = (c0 + j) * K
    pltpu.async_copy(src1.at[pl.ds(eb, K)], sb, si)
    pltpu.async_copy(dst1.at[pl.ds(eb, K)], db, si)

  def iw(j, sb, db, si):
    eb = (c0 + j) * K
    pltpu.make_async_copy(src1.at[pl.ds(eb, K)], sb, si).wait()
    pltpu.make_async_copy(dst1.at[pl.ds(eb, K)], db, si).wait()

  def gs(sb, r, sg):
    pltpu.async_copy(table.at[sb], r, sg)

  def gw(sb, r, sg):
    pltpu.make_async_copy(table.at[sb], r, sg).wait()

  def sc(r, db):
    pltpu.sync_copy(r, acc.at[db], add=True)

  def pair(j0, pre2, pre3):
    gw(sb0, r0, sg0)
    iw(j0 + 1, sb1, db1, si1)
    gs(sb1, r1, sg1)
    sc(r0, db0)
    if pre2:
      ist(j0 + 2, sb0, db0, si0)
    gw(sb1, r1, sg1)
    if pre2:
      iw(j0 + 2, sb0, db0, si0)
      gs(sb0, r0, sg0)
    sc(r1, db1)
    if pre3:
      ist(j0 + 3, sb1, db1, si1)

  ist(0, sb0, db0, si0)
  iw(0, sb0, db0, si0)
  gs(sb0, r0, sg0)
  ist(1, sb1, db1, si1)

  nbody = nch // 2 - 1

  def body(i, carry):
    pair(2 * i, True, True)
    return carry

  lax.fori_loop(0, nbody, body, 0)

  if nch % 2 == 0:
    pair(nch - 2, False, False)
  else:
    pair(nch - 3, True, False)
    gw(sb0, r0, sg0)
    sc(r0, db0)


_SCATTER_SCRATCH = [
    pltpu.VMEM_SHARED((NP, FD), f32),
    pltpu.VMEM((STG, FD), f32),
    pltpu.VMEM((K,), jnp.int32),
    pltpu.VMEM((K,), jnp.int32),
    pltpu.VMEM((K,), jnp.int32),
    pltpu.VMEM((K,), jnp.int32),
    pltpu.VMEM((K, FD), f32),
    pltpu.VMEM((K, FD), f32),
    pltpu.SemaphoreType.DMA,
    pltpu.SemaphoreType.DMA,
    pltpu.SemaphoreType.DMA,
    pltpu.SemaphoreType.DMA,
]


def _pair_body(tblA, srcA, dstA, tblB, srcB, dstB, zeros_h,
               outA, outB,
               acc, zb, sb0, sb1, db0, db1, r0, r1, si0, si1, sg0, sg1):
  c = lax.axis_index("c")
  s = lax.axis_index("s")
  rb = s * RPS
  nch = (NE // K) // NSUB

  @pl.when(c == 0)
  def _():
    _stage_rows(tblA, acc, rb, zb)

  @pl.when(c == 1)
  def _():
    _fill_rows(zeros_h, acc, rb, zb)

  plsc.subcore_barrier()

  @pl.when(c == 0)
  def _():
    _scatter_stream(nch, s * nch, tblA, srcA, dstA, acc,
                    sb0, sb1, db0, db1, r0, r1, si0, si1, sg0, sg1)

  @pl.when(c == 1)
  def _():
    _scatter_stream(nch, s * nch, tblB, srcB, dstB, acc,
                    sb0, sb1, db0, db1, r0, r1, si0, si1, sg0, sg1)

  plsc.subcore_barrier()

  @pl.when(c == 0)
  def _():
    _stage_rows(acc, outA, rb, zb)

  @pl.when(c == 1)
  def _():
    _stage_rows(acc, outB, rb, zb)


_pair_pass = pl.kernel(
    _pair_body,
    out_type=(jax.ShapeDtypeStruct((NP, FD), f32),
              jax.ShapeDtypeStruct((NP, FD), f32)),
    mesh=_MESH,
    scratch_types=list(_SCATTER_SCRATCH),
    name="sc_scatter_pair",
)


def _solo_body(tbl, src1, dst1, zeros_h,
               outA, outB,
               acc, zb, sb0, sb1, db0, db1, r0, r1, si0, si1, sg0, sg1):
  c = lax.axis_index("c")
  s = lax.axis_index("s")
  rb = s * RPS
  nch = (NE // K) // (NSUB * NCORE)

  _fill_rows(zeros_h, acc, rb, zb)
  plsc.subcore_barrier()

  @pl.when(c == 0)
  def _():
    _scatter_stream(nch, s * nch, tbl, src1, dst1, acc,
                    sb0, sb1, db0, db1, r0, r1, si0, si1, sg0, sg1)

  @pl.when(c == 1)
  def _():
    _scatter_stream(nch, NSUB * nch + s * nch, tbl, src1, dst1, acc,
                    sb0, sb1, db0, db1, r0, r1, si0, si1, sg0, sg1)

  plsc.subcore_barrier()

  @pl.when(c == 0)
  def _():
    _stage_rows(acc, outA, rb, zb)

  @pl.when(c == 1)
  def _():
    _stage_rows(acc, outB, rb, zb)


_solo_pass = pl.kernel(
    _solo_body,
    out_type=(jax.ShapeDtypeStruct((NP, FD), f32),
              jax.ShapeDtypeStruct((NP, FD), f32)),
    mesh=_MESH,
    scratch_types=list(_SCATTER_SCRATCH),
    name="sc_scatter_solo",
)


def _count_body(dst1, ni1, ei1, ones_d, ones_n, ones_e, zeros_h,
                out0, out1,
                acc, ones_v, zb, db0, db1, si0, si1):
  c = lax.axis_index("c")
  s = lax.axis_index("s")
  rb = s * RPS
  nch = (NE // K) // (NSUB * NCORE)

  _fill_rows(zeros_h, acc, rb, zb)
  plsc.subcore_barrier()

  def run_array(idx1, ones_x, cbase):
    pltpu.sync_copy(ones_x, ones_v)
    c0 = cbase + s * nch

    def ist(j, db, si):
      pltpu.async_copy(idx1.at[pl.ds((c0 + j) * K, K)], db, si)

    def iw(j, db, si):
      pltpu.make_async_copy(idx1.at[pl.ds((c0 + j) * K, K)], db, si).wait()

    def sc(db):
      pltpu.sync_copy(ones_v, acc.at[db], add=True)

    ist(0, db0, si0)
    ist(1, db1, si1)

    def body(i, carry):
      j0 = 2 * i
      iw(j0, db0, si0)
      sc(db0)
      ist(j0 + 2, db0, si0)
      iw(j0 + 1, db1, si1)
      sc(db1)
      ist(j0 + 3, db1, si1)
      return carry

    lax.fori_loop(0, nch // 2 - 2, body, 0)
    iw(nch - 5, db0, si0)
    sc(db0)
    ist(nch - 3, db0, si0)
    iw(nch - 4, db1, si1)
    sc(db1)
    ist(nch - 2, db1, si1)
    iw(nch - 3, db0, si0)
    sc(db0)
    ist(nch - 1, db0, si0)
    iw(nch - 2, db1, si1)
    sc(db1)
    iw(nch - 1, db0, si0)
    sc(db0)

  for idx1, ones_x in ((dst1, ones_d), (ni1, ones_n), (ei1, ones_e)):
    @pl.when(c == 0)
    def _(idx1=idx1, ones_x=ones_x):
      run_array(idx1, ones_x, 0)

    @pl.when(c == 1)
    def _(idx1=idx1, ones_x=ones_x):
      run_array(idx1, ones_x, NSUB * nch)

  plsc.subcore_barrier()

  @pl.when(c == 0)
  def _():
    _stage_rows(acc, out0, rb, zb)

  @pl.when(c == 1)
  def _():
    _stage_rows(acc, out1, rb, zb)


_count_pass = pl.kernel(
    _count_body,
    out_type=(jax.ShapeDtypeStruct((NP, FD), f32),
              jax.ShapeDtypeStruct((NP, FD), f32)),
    mesh=_MESH,
    scratch_types=[
        pltpu.VMEM_SHARED((NP, FD), f32),
        pltpu.VMEM((K, FD), f32),
        pltpu.VMEM((STG, FD), f32),
        pltpu.VMEM((K,), jnp.int32),
        pltpu.VMEM((K,), jnp.int32),
        pltpu.SemaphoreType.DMA,
        pltpu.SemaphoreType.DMA,
    ],
    name="sc_counts",
)


BT = 2048
GRID = NP // BT

_row = pl.BlockSpec((BT, FD), lambda i: (i, 0))
_wgt = pl.BlockSpec((FD, FD), lambda i: (0, 0))
_bia = pl.BlockSpec((1, FD), lambda i: (0, 0))
_gat = pl.BlockSpec((1, 1), lambda i: (0, 0))

_CDST, _CNI, _CEI = 0, 1, 2


def _elu(v):
  return jnp.where(v > 0, v, jnp.exp(jnp.minimum(v, 0.0)) - 1.0)


def _dinv_gcn(c0, c1):
  t = c0[...][:, _CDST:_CDST + 1] + c1[...][:, _CDST:_CDST + 1]
  return lax.rsqrt(1.0 + t)


def _inv_pos(c0, c1, lane):
  t = c0[...][:, lane:lane + 1] + c1[...][:, lane:lane + 1]
  return jnp.where(t > 0, 1.0 / t, 0.0)


def _prep1_body(x_r, wg_r, wh_r, ct0_r, ct1_r, y1_r, xwh_r):
  dinv = _dinv_gcn(ct0_r, ct1_r)
  xb = x_r[...]
  y1_r[...] = dinv * jnp.dot(xb, wg_r[...], preferred_element_type=f32)
  xwh_r[...] = jnp.dot(xb, wh_r[...], preferred_element_type=f32)


_prep1 = pl.pallas_call(
    _prep1_body,
    grid=(GRID,),
    in_specs=[_row, _wgt, _wgt, _row, _row],
    out_specs=(_row, _row),
    out_shape=(jax.ShapeDtypeStruct((NP, FD), f32),
               jax.ShapeDtypeStruct((NP, FD), f32)),
    name="tc_prep1",
)


def _mid1_body(accG_r, accH_r, ct0_r, ct1_r, bg1_r, wg2_r,
               y2_r, m1_r):
  dinv = _dinv_gcn(ct0_r, ct1_r)
  xs1 = _elu(dinv * accG_r[...] + bg1_r[...])
  y2_r[...] = dinv * jnp.dot(xs1, wg2_r[...], preferred_element_type=f32)
  m1_r[...] = _inv_pos(ct0_r, ct1_r, _CEI) * accH_r[...]


_mid1 = pl.pallas_call(
    _mid1_body,
    grid=(GRID,),
    in_specs=[_row, _row, _row, _row, _bia, _wgt],
    out_specs=(_row, _row),
    out_shape=(jax.ShapeDtypeStruct((NP, FD), f32),
               jax.ShapeDtypeStruct((NP, FD), f32)),
    name="tc_mid1",
)


def _mid2_body(accG2_r, accH2_r, ct0_r, ct1_r, bg2_r, bh1_r,
               wh2_r, xs_r, yh2_r):
  dinv = _dinv_gcn(ct0_r, ct1_r)
  xs_r[...] = dinv * accG2_r[...] + bg2_r[...]
  xd1 = _elu(_inv_pos(ct0_r, ct1_r, _CNI) * accH2_r[...] + bh1_r[...])
  yh2_r[...] = jnp.dot(xd1, wh2_r[...], preferred_element_type=f32)


_mid2 = pl.pallas_call(
    _mid2_body,
    grid=(GRID,),
    in_specs=[_row, _row, _row, _row, _bia, _bia, _wgt],
    out_specs=(_row, _row),
    out_shape=(jax.ShapeDtypeStruct((NP, FD), f32),
               jax.ShapeDtypeStruct((NP, FD), f32)),
    name="tc_mid2",
)


def _mid3_body(a_r, b_r, ct0_r, ct1_r, m2_r):
  m2_r[...] = _inv_pos(ct0_r, ct1_r, _CEI) * (a_r[...] + b_r[...])


_mid3 = pl.pallas_call(
    _mid3_body,
    grid=(GRID,),
    in_specs=[_row, _row, _row, _row],
    out_specs=_row,
    out_shape=jax.ShapeDtypeStruct((NP, FD), f32),
    name="tc_mid3",
)


def _final_body(a_r, b_r, ct0_r, ct1_r, bh2_r, xs_r, g_r, z_r, xd_r):
  xd = _inv_pos(ct0_r, ct1_r, _CNI) * (a_r[...] + b_r[...]) + bh2_r[...]
  xd_r[...] = xd
  alpha = 1.0 / (1.0 + jnp.exp(-g_r[...]))
  z_r[...] = alpha * xs_r[...] + (1.0 - alpha) * xd


_final = pl.pallas_call(
    _final_body,
    grid=(GRID,),
    in_specs=[_row, _row, _row, _row, _bia, _row, _gat],
    out_specs=(_row, _row),
    out_shape=(jax.ShapeDtypeStruct((NP, FD), f32),
               jax.ShapeDtypeStruct((NP, FD), f32)),
    name="tc_final",
)


def kernel(x, edge_index, hyperedge_index,
           W_g1, b_g1, W_g2, b_g2, W_h1, b_h1, W_h2, b_h2, gate):
  src1 = edge_index[0]
  dst1 = edge_index[1]
  ni1 = hyperedge_index[0]
  ei1 = hyperedge_index[1]

  xp = jnp.pad(x, ((0, NP - NN), (0, 0)))
  zeros128 = jnp.zeros((STG, FD), f32)
  eye = jnp.eye(3, FD, dtype=f32)
  ones_d = jnp.tile(eye[0][None], (K, 1))
  ones_n = jnp.tile(eye[1][None], (K, 1))
  ones_e = jnp.tile(eye[2][None], (K, 1))
  bg1 = b_g1.reshape(1, FD)
  bg2 = b_g2.reshape(1, FD)
  bh1 = b_h1.reshape(1, FD)
  bh2 = b_h2.reshape(1, FD)
  g2 = gate.reshape(1, 1)

  ct0, ct1 = _count_pass(dst1, ni1, ei1, ones_d, ones_n, ones_e, zeros128)

  y1, xwh1 = _prep1(xp, W_g1, W_h1, ct0, ct1)
  accG, accH = _pair_pass(y1, src1, dst1, xwh1, ni1, ei1, zeros128)
  y2, m1 = _mid1(accG, accH, ct0, ct1, bg1, W_g2)
  accG2, accH2 = _pair_pass(y2, src1, dst1, m1, ei1, ni1, zeros128)
  xs, yh2 = _mid2(accG2, accH2, ct0, ct1, bg2, bh1, W_h2)
  a3, b3 = _solo_pass(yh2, ni1, ei1, zeros128)
  m2 = _mid3(a3, b3, ct0, ct1)
  a4, b4 = _solo_pass(m2, ei1, ni1, zeros128)
  zf, xdf = _final(a4, b4, ct0, ct1, bh2, xs, g2)
  return (zf[:NN], xs[:NN], xdf[:NN])


# speedup vs baseline: 17.1755x; 1.1883x over previous
import jax
import jax.numpy as jnp
from jax import lax
from jax.experimental import pallas as pl
from jax.experimental.pallas import tpu as pltpu
from jax.experimental.pallas import tpu_sc as plsc

f32 = jnp.float32

NN = 10000
NP = 10240
FD = 128
NE = 320000
K = 80
NSUB = 16
NCORE = 2
RPS = NP // NSUB
STG = 80
CW = 16

_MESH = plsc.VectorSubcoreMesh(
    core_axis_name="c", subcore_axis_name="s",
    num_cores=NCORE, num_subcores=NSUB)


def _stage_rows(src, dst, base, zb):
  for t in range(RPS // STG):
    pltpu.sync_copy(src.at[pl.ds(base + t * STG, STG)], zb)
    pltpu.sync_copy(zb, dst.at[pl.ds(base + t * STG, STG)])


def _fill_rows(zsrc, dst, base, zb):
  pltpu.sync_copy(zsrc, zb)
  for t in range(RPS // STG):
    pltpu.sync_copy(zb, dst.at[pl.ds(base + t * STG, STG)])


_NSLOT = 4


def _scatter_stream(nch, c0, table, src1, dst1, acc, sb, db, r, si, sg, ss):
  def ist(j, t):
    eb = (c0 + j) * K
    pltpu.async_copy(src1.at[pl.ds(eb, K)], sb[t], si[t])
    pltpu.async_copy(dst1.at[pl.ds(eb, K)], db[t], si[t])

  def iw(j, t):
    eb = (c0 + j) * K
    pltpu.make_async_copy(src1.at[pl.ds(eb, K)], sb[t], si[t]).wait()
    pltpu.make_async_copy(dst1.at[pl.ds(eb, K)], db[t], si[t]).wait()

  def gs(t):
    pltpu.async_copy(table.at[sb[t]], r[t], sg[t])

  def gw(t):
    pltpu.make_async_copy(table.at[sb[t]], r[t], sg[t]).wait()

  def scs(t):
    pltpu.async_copy(r[t], acc.at[db[t]], ss[t], add=True)

  def scw(t):
    pltpu.make_async_copy(r[t], acc.at[db[t]], ss[t]).wait()

  def step(j, t, nxt, ssw, pref):
    if nxt:
      tn = (t + 1) % _NSLOT
      iw(j + 1, tn)
      gs(tn)
    gw(t)
    scs(t)
    if ssw:
      scw((t + 2) % _NSLOT)
    if pref:
      ist(j + 2, (t + 2) % _NSLOT)

  ist(0, 0)
  ist(1, 1)
  iw(0, 0)
  gs(0)

  step(0, 0, True, False, True)
  step(1, 1, True, False, True)

  ngrp = (nch - 4) // 4

  def body(g, carry):
    j0 = 4 * g + 2
    for t in range(_NSLOT):
      step(j0 + t, (2 + t) % _NSLOT, True, True, True)
    return carry

  lax.fori_loop(0, ngrp, body, 0)

  for j in range(4 * ngrp + 2, nch):
    step(j, j % _NSLOT, j + 1 < nch, True, j + 2 < nch)

  scw((nch - 2) % _NSLOT)
  scw((nch - 1) % _NSLOT)


_SCATTER_SCRATCH = (
    [pltpu.VMEM_SHARED((NP, FD), f32)]
    + [pltpu.VMEM((K,), jnp.int32) for _ in range(2 * _NSLOT)]
    + [pltpu.VMEM((K, FD), f32) for _ in range(_NSLOT)]
    + [pltpu.SemaphoreType.DMA for _ in range(3 * _NSLOT)]
)


def _pair_body(tblA, srcA, dstA, tblB, srcB, dstB, zeros_h,
               outA, outB, acc, *slots):
  sb, db, r, si, sg, ss = (slots[0:4], slots[4:8], slots[8:12],
                           slots[12:16], slots[16:20], slots[20:24])
  zb = r[0]
  c = lax.axis_index("c")
  s = lax.axis_index("s")
  rb = s * RPS
  nch = (NE // K) // NSUB

  @pl.when(c == 0)
  def _():
    _stage_rows(tblA, acc, rb, zb)

  @pl.when(c == 1)
  def _():
    _fill_rows(zeros_h, acc, rb, zb)

  plsc.subcore_barrier()

  @pl.when(c == 0)
  def _():
    _scatter_stream(nch, s * nch, tblA, srcA, dstA, acc, sb, db, r, si, sg, ss)

  @pl.when(c == 1)
  def _():
    _scatter_stream(nch, s * nch, tblB, srcB, dstB, acc, sb, db, r, si, sg, ss)

  plsc.subcore_barrier()

  @pl.when(c == 0)
  def _():
    _stage_rows(acc, outA, rb, zb)

  @pl.when(c == 1)
  def _():
    _stage_rows(acc, outB, rb, zb)


_pair_pass = pl.kernel(
    _pair_body,
    out_type=(jax.ShapeDtypeStruct((NP, FD), f32),
              jax.ShapeDtypeStruct((NP, FD), f32)),
    mesh=_MESH,
    scratch_types=list(_SCATTER_SCRATCH),
    name="sc_scatter_pair",
)


def _solo_body(tbl, src1, dst1, zeros_h,
               outA, outB, acc, *slots):
  sb, db, r, si, sg, ss = (slots[0:4], slots[4:8], slots[8:12],
                           slots[12:16], slots[16:20], slots[20:24])
  zb = r[0]
  c = lax.axis_index("c")
  s = lax.axis_index("s")
  rb = s * RPS
  nch = (NE // K) // (NSUB * NCORE)

  _fill_rows(zeros_h, acc, rb, zb)
  plsc.subcore_barrier()

  @pl.when(c == 0)
  def _():
    _scatter_stream(nch, s * nch, tbl, src1, dst1, acc, sb, db, r, si, sg, ss)

  @pl.when(c == 1)
  def _():
    _scatter_stream(nch, NSUB * nch + s * nch, tbl, src1, dst1, acc,
                    sb, db, r, si, sg, ss)

  plsc.subcore_barrier()

  @pl.when(c == 0)
  def _():
    _stage_rows(acc, outA, rb, zb)

  @pl.when(c == 1)
  def _():
    _stage_rows(acc, outB, rb, zb)


_solo_pass = pl.kernel(
    _solo_body,
    out_type=(jax.ShapeDtypeStruct((NP, FD), f32),
              jax.ShapeDtypeStruct((NP, FD), f32)),
    mesh=_MESH,
    scratch_types=list(_SCATTER_SCRATCH),
    name="sc_scatter_solo",
)


def _count_body(dst1, ni1, ei1, ones_d, ones_n, ones_e, zeros_h,
                out0, out1,
                acc, ones_v, zb, db0, db1, si0, si1):
  c = lax.axis_index("c")
  s = lax.axis_index("s")
  rb = s * RPS
  nch = (NE // K) // (NSUB * NCORE)

  _fill_rows(zeros_h, acc, rb, zb)
  plsc.subcore_barrier()

  def run_array(idx1, ones_x, cbase):
    pltpu.sync_copy(ones_x, ones_v)
    c0 = cbase + s * nch

    def ist(j, db, si):
      pltpu.async_copy(idx1.at[pl.ds((c0 + j) * K, K)], db, si)

    def iw(j, db, si):
      pltpu.make_async_copy(idx1.at[pl.ds((c0 + j) * K, K)], db, si).wait()

    def sc(db):
      pltpu.sync_copy(ones_v, acc.at[db], add=True)

    ist(0, db0, si0)
    ist(1, db1, si1)

    def body(i, carry):
      j0 = 2 * i
      iw(j0, db0, si0)
      sc(db0)
      ist(j0 + 2, db0, si0)
      iw(j0 + 1, db1, si1)
      sc(db1)
      ist(j0 + 3, db1, si1)
      return carry

    lax.fori_loop(0, nch // 2 - 2, body, 0)
    iw(nch - 5, db0, si0)
    sc(db0)
    ist(nch - 3, db0, si0)
    iw(nch - 4, db1, si1)
    sc(db1)
    ist(nch - 2, db1, si1)
    iw(nch - 3, db0, si0)
    sc(db0)
    ist(nch - 1, db0, si0)
    iw(nch - 2, db1, si1)
    sc(db1)
    iw(nch - 1, db0, si0)
    sc(db0)

  for idx1, ones_x in ((dst1, ones_d), (ni1, ones_n), (ei1, ones_e)):
    @pl.when(c == 0)
    def _(idx1=idx1, ones_x=ones_x):
      run_array(idx1, ones_x, 0)

    @pl.when(c == 1)
    def _(idx1=idx1, ones_x=ones_x):
      run_array(idx1, ones_x, NSUB * nch)

  plsc.subcore_barrier()

  @pl.when(c == 0)
  def _():
    _stage_rows(acc, out0, rb, zb)

  @pl.when(c == 1)
  def _():
    _stage_rows(acc, out1, rb, zb)


_count_pass = pl.kernel(
    _count_body,
    out_type=(jax.ShapeDtypeStruct((NP, FD), f32),
              jax.ShapeDtypeStruct((NP, FD), f32)),
    mesh=_MESH,
    scratch_types=[
        pltpu.VMEM_SHARED((NP, FD), f32),
        pltpu.VMEM((K, FD), f32),
        pltpu.VMEM((STG, FD), f32),
        pltpu.VMEM((K,), jnp.int32),
        pltpu.VMEM((K,), jnp.int32),
        pltpu.SemaphoreType.DMA,
        pltpu.SemaphoreType.DMA,
    ],
    name="sc_counts",
)


BT = 2048
GRID = NP // BT

_row = pl.BlockSpec((BT, FD), lambda i: (i, 0))
_wgt = pl.BlockSpec((FD, FD), lambda i: (0, 0))
_bia = pl.BlockSpec((1, FD), lambda i: (0, 0))
_gat = pl.BlockSpec((1, 1), lambda i: (0, 0))

_CDST, _CNI, _CEI = 0, 1, 2


def _elu(v):
  return jnp.where(v > 0, v, jnp.exp(jnp.minimum(v, 0.0)) - 1.0)


def _dinv_gcn(c0, c1):
  t = c0[...][:, _CDST:_CDST + 1] + c1[...][:, _CDST:_CDST + 1]
  return lax.rsqrt(1.0 + t)


def _inv_pos(c0, c1, lane):
  t = c0[...][:, lane:lane + 1] + c1[...][:, lane:lane + 1]
  return jnp.where(t > 0, 1.0 / t, 0.0)


def _prep1_body(x_r, wg_r, wh_r, ct0_r, ct1_r, y1_r, xwh_r):
  dinv = _dinv_gcn(ct0_r, ct1_r)
  xb = x_r[...]
  y1_r[...] = dinv * jnp.dot(xb, wg_r[...], preferred_element_type=f32)
  xwh_r[...] = jnp.dot(xb, wh_r[...], preferred_element_type=f32)


_prep1 = pl.pallas_call(
    _prep1_body,
    grid=(GRID,),
    in_specs=[_row, _wgt, _wgt, _row, _row],
    out_specs=(_row, _row),
    out_shape=(jax.ShapeDtypeStruct((NP, FD), f32),
               jax.ShapeDtypeStruct((NP, FD), f32)),
    name="tc_prep1",
)


def _mid1_body(accG_r, accH_r, ct0_r, ct1_r, bg1_r, wg2_r,
               y2_r, m1_r):
  dinv = _dinv_gcn(ct0_r, ct1_r)
  xs1 = _elu(dinv * accG_r[...] + bg1_r[...])
  y2_r[...] = dinv * jnp.dot(xs1, wg2_r[...], preferred_element_type=f32)
  m1_r[...] = _inv_pos(ct0_r, ct1_r, _CEI) * accH_r[...]


_mid1 = pl.pallas_call(
    _mid1_body,
    grid=(GRID,),
    in_specs=[_row, _row, _row, _row, _bia, _wgt],
    out_specs=(_row, _row),
    out_shape=(jax.ShapeDtypeStruct((NP, FD), f32),
               jax.ShapeDtypeStruct((NP, FD), f32)),
    name="tc_mid1",
)


def _mid2_body(accG2_r, accH2_r, ct0_r, ct1_r, bg2_r, bh1_r,
               wh2_r, xs_r, yh2_r):
  dinv = _dinv_gcn(ct0_r, ct1_r)
  xs_r[...] = dinv * accG2_r[...] + bg2_r[...]
  xd1 = _elu(_inv_pos(ct0_r, ct1_r, _CNI) * accH2_r[...] + bh1_r[...])
  yh2_r[...] = jnp.dot(xd1, wh2_r[...], preferred_element_type=f32)


_mid2 = pl.pallas_call(
    _mid2_body,
    grid=(GRID,),
    in_specs=[_row, _row, _row, _row, _bia, _bia, _wgt],
    out_specs=(_row, _row),
    out_shape=(jax.ShapeDtypeStruct((NP, FD), f32),
               jax.ShapeDtypeStruct((NP, FD), f32)),
    name="tc_mid2",
)


def _mid3_body(a_r, b_r, ct0_r, ct1_r, m2_r):
  m2_r[...] = _inv_pos(ct0_r, ct1_r, _CEI) * (a_r[...] + b_r[...])


_mid3 = pl.pallas_call(
    _mid3_body,
    grid=(GRID,),
    in_specs=[_row, _row, _row, _row],
    out_specs=_row,
    out_shape=jax.ShapeDtypeStruct((NP, FD), f32),
    name="tc_mid3",
)


def _final_body(a_r, b_r, ct0_r, ct1_r, bh2_r, xs_r, g_r, z_r, xd_r):
  xd = _inv_pos(ct0_r, ct1_r, _CNI) * (a_r[...] + b_r[...]) + bh2_r[...]
  xd_r[...] = xd
  alpha = 1.0 / (1.0 + jnp.exp(-g_r[...]))
  z_r[...] = alpha * xs_r[...] + (1.0 - alpha) * xd


_final = pl.pallas_call(
    _final_body,
    grid=(GRID,),
    in_specs=[_row, _row, _row, _row, _bia, _row, _gat],
    out_specs=(_row, _row),
    out_shape=(jax.ShapeDtypeStruct((NP, FD), f32),
               jax.ShapeDtypeStruct((NP, FD), f32)),
    name="tc_final",
)


def kernel(x, edge_index, hyperedge_index,
           W_g1, b_g1, W_g2, b_g2, W_h1, b_h1, W_h2, b_h2, gate):
  src1 = edge_index[0]
  dst1 = edge_index[1]
  ni1 = hyperedge_index[0]
  ei1 = hyperedge_index[1]

  xp = jnp.pad(x, ((0, NP - NN), (0, 0)))
  zeros128 = jnp.zeros((STG, FD), f32)
  eye = jnp.eye(3, FD, dtype=f32)
  ones_d = jnp.tile(eye[0][None], (K, 1))
  ones_n = jnp.tile(eye[1][None], (K, 1))
  ones_e = jnp.tile(eye[2][None], (K, 1))
  bg1 = b_g1.reshape(1, FD)
  bg2 = b_g2.reshape(1, FD)
  bh1 = b_h1.reshape(1, FD)
  bh2 = b_h2.reshape(1, FD)
  g2 = gate.reshape(1, 1)

  ct0, ct1 = _count_pass(dst1, ni1, ei1, ones_d, ones_n, ones_e, zeros128)

  y1, xwh1 = _prep1(xp, W_g1, W_h1, ct0, ct1)
  accG, accH = _pair_pass(y1, src1, dst1, xwh1, ni1, ei1, zeros128)
  y2, m1 = _mid1(accG, accH, ct0, ct1, bg1, W_g2)
  accG2, accH2 = _pair_pass(y2, src1, dst1, m1, ei1, ni1, zeros128)
  xs, yh2 = _mid2(accG2, accH2, ct0, ct1, bg2, bh1, W_h2)
  a3, b3 = _solo_pass(yh2, ni1, ei1, zeros128)
  m2 = _mid3(a3, b3, ct0, ct1)
  a4, b4 = _solo_pass(m2, ei1, ni1, zeros128)
  zf, xdf = _final(a4, b4, ct0, ct1, bh2, xs, g2)
  return (zf[:NN], xs[:NN], xdf[:NN])


# speedup vs baseline: 17.3156x; 1.0082x over previous
import jax
import jax.numpy as jnp
from jax import lax
from jax.experimental import pallas as pl
from jax.experimental.pallas import tpu as pltpu
from jax.experimental.pallas import tpu_sc as plsc

f32 = jnp.float32

NN = 10000
NP = 10240
FD = 128
NE = 320000
K = 80
NSUB = 16
NCORE = 2
RPS = NP // NSUB
STG = 80
CW = 16

_MESH = plsc.VectorSubcoreMesh(
    core_axis_name="c", subcore_axis_name="s",
    num_cores=NCORE, num_subcores=NSUB)


def _stage_rows(src, dst, base, zb):
  for t in range(RPS // STG):
    pltpu.sync_copy(src.at[pl.ds(base + t * STG, STG)], zb)
    pltpu.sync_copy(zb, dst.at[pl.ds(base + t * STG, STG)])


def _fill_rows(zsrc, dst, base, zb):
  pltpu.sync_copy(zsrc, zb)
  for t in range(RPS // STG):
    pltpu.sync_copy(zb, dst.at[pl.ds(base + t * STG, STG)])


_NSLOT = 4


def _scatter_stream(nch, c0, table, src1, dst1, acc, sb, db, r, si, sg, ss):
  def ist(j, t):
    eb = (c0 + j) * K
    pltpu.async_copy(src1.at[pl.ds(eb, K)], sb[t], si[t])
    pltpu.async_copy(dst1.at[pl.ds(eb, K)], db[t], si[t])

  def iw(j, t):
    eb = (c0 + j) * K
    pltpu.make_async_copy(src1.at[pl.ds(eb, K)], sb[t], si[t]).wait()
    pltpu.make_async_copy(dst1.at[pl.ds(eb, K)], db[t], si[t]).wait()

  def gs(t):
    pltpu.async_copy(table.at[sb[t]], r[t], sg[t])

  def gw(t):
    pltpu.make_async_copy(table.at[sb[t]], r[t], sg[t]).wait()

  def scs(t):
    pltpu.async_copy(r[t], acc.at[db[t]], ss[t], add=True)

  def scw(t):
    pltpu.make_async_copy(r[t], acc.at[db[t]], ss[t]).wait()

  def step(j, t, nxt, ssw, pref):
    if nxt:
      tn = (t + 1) % _NSLOT
      iw(j + 1, tn)
      gs(tn)
    gw(t)
    scs(t)
    if ssw:
      scw((t + 2) % _NSLOT)
    if pref:
      ist(j + 2, (t + 2) % _NSLOT)

  ist(0, 0)
  ist(1, 1)
  iw(0, 0)
  gs(0)

  step(0, 0, True, False, True)
  step(1, 1, True, False, True)

  ngrp = (nch - 4) // 4

  def body(g, carry):
    j0 = 4 * g + 2
    for t in range(_NSLOT):
      step(j0 + t, (2 + t) % _NSLOT, True, True, True)
    return carry

  lax.fori_loop(0, ngrp, body, 0)

  for j in range(4 * ngrp + 2, nch):
    step(j, j % _NSLOT, j + 1 < nch, True, j + 2 < nch)

  scw((nch - 2) % _NSLOT)
  scw((nch - 1) % _NSLOT)


_SCATTER_SCRATCH = (
    [pltpu.VMEM_SHARED((NP, FD), f32)]
    + [pltpu.VMEM((K,), jnp.int32) for _ in range(2 * _NSLOT)]
    + [pltpu.VMEM((K, FD), f32) for _ in range(_NSLOT)]
    + [pltpu.SemaphoreType.DMA for _ in range(3 * _NSLOT)]
)


def _pair_body(tblA, srcA, dstA, tblB, srcB, dstB, zeros_h,
               outA, outB, acc, *slots):
  sb, db, r, si, sg, ss = (slots[0:4], slots[4:8], slots[8:12],
                           slots[12:16], slots[16:20], slots[20:24])
  zb = r[0]
  c = lax.axis_index("c")
  s = lax.axis_index("s")
  rb = s * RPS
  nch = (NE // K) // NSUB

  @pl.when(c == 0)
  def _():
    _stage_rows(tblA, acc, rb, zb)

  @pl.when(c == 1)
  def _():
    _fill_rows(zeros_h, acc, rb, zb)

  plsc.subcore_barrier()

  @pl.when(c == 0)
  def _():
    _scatter_stream(nch, s * nch, tblA, srcA, dstA, acc, sb, db, r, si, sg, ss)

  @pl.when(c == 1)
  def _():
    _scatter_stream(nch, s * nch, tblB, srcB, dstB, acc, sb, db, r, si, sg, ss)

  plsc.subcore_barrier()

  @pl.when(c == 0)
  def _():
    _stage_rows(acc, outA, rb, zb)

  @pl.when(c == 1)
  def _():
    _stage_rows(acc, outB, rb, zb)


_pair_pass = pl.kernel(
    _pair_body,
    out_type=(jax.ShapeDtypeStruct((NP, FD), f32),
              jax.ShapeDtypeStruct((NP, FD), f32)),
    mesh=_MESH,
    scratch_types=list(_SCATTER_SCRATCH),
    name="sc_scatter_pair",
)


def _solo_body(tbl, src1, dst1, zeros_h,
               outA, outB, acc, *slots):
  sb, db, r, si, sg, ss = (slots[0:4], slots[4:8], slots[8:12],
                           slots[12:16], slots[16:20], slots[20:24])
  zb = r[0]
  c = lax.axis_index("c")
  s = lax.axis_index("s")
  rb = s * RPS
  nch = (NE // K) // (NSUB * NCORE)

  _fill_rows(zeros_h, acc, rb, zb)
  plsc.subcore_barrier()

  @pl.when(c == 0)
  def _():
    _scatter_stream(nch, s * nch, tbl, src1, dst1, acc, sb, db, r, si, sg, ss)

  @pl.when(c == 1)
  def _():
    _scatter_stream(nch, NSUB * nch + s * nch, tbl, src1, dst1, acc,
                    sb, db, r, si, sg, ss)

  plsc.subcore_barrier()

  @pl.when(c == 0)
  def _():
    _stage_rows(acc, outA, rb, zb)

  @pl.when(c == 1)
  def _():
    _stage_rows(acc, outB, rb, zb)


_solo_pass = pl.kernel(
    _solo_body,
    out_type=(jax.ShapeDtypeStruct((NP, FD), f32),
              jax.ShapeDtypeStruct((NP, FD), f32)),
    mesh=_MESH,
    scratch_types=list(_SCATTER_SCRATCH),
    name="sc_scatter_solo",
)


def _count_body(dst1, ni1, ei1, ones_d, ones_n, ones_e, zeros_h,
                out0, out1, acc, ones_v, zb, *slots):
  db, si, ss = slots[0:4], slots[4:8], slots[8:12]
  c = lax.axis_index("c")
  s = lax.axis_index("s")
  rb = s * RPS
  nch = (NE // K) // (NSUB * NCORE)

  _fill_rows(zeros_h, acc, rb, zb)
  plsc.subcore_barrier()

  def run_array(idx1, ones_x, cbase):
    pltpu.sync_copy(ones_x, ones_v)
    c0 = cbase + s * nch

    def ist(j, t):
      pltpu.async_copy(idx1.at[pl.ds((c0 + j) * K, K)], db[t], si[t])

    def iw(j, t):
      pltpu.make_async_copy(
          idx1.at[pl.ds((c0 + j) * K, K)], db[t], si[t]).wait()

    def scs(t):
      pltpu.async_copy(ones_v, acc.at[db[t]], ss[t], add=True)

    def scw(t):
      pltpu.make_async_copy(ones_v, acc.at[db[t]], ss[t]).wait()

    def step(j, t, ssw, pref):
      iw(j, t)
      scs(t)
      if ssw:
        scw((t + 2) % _NSLOT)
      if pref:
        ist(j + 2, (t + 2) % _NSLOT)

    ist(0, 0)
    ist(1, 1)
    step(0, 0, False, True)
    step(1, 1, False, True)

    ngrp = (nch - 4) // 4

    def body(g, carry):
      j0 = 4 * g + 2
      for t in range(_NSLOT):
        step(j0 + t, (2 + t) % _NSLOT, True, True)
      return carry

    lax.fori_loop(0, ngrp, body, 0)

    for j in range(4 * ngrp + 2, nch):
      step(j, j % _NSLOT, True, j + 2 < nch)

    scw((nch - 2) % _NSLOT)
    scw((nch - 1) % _NSLOT)

  for idx1, ones_x in ((dst1, ones_d), (ni1, ones_n), (ei1, ones_e)):
    @pl.when(c == 0)
    def _(idx1=idx1, ones_x=ones_x):
      run_array(idx1, ones_x, 0)

    @pl.when(c == 1)
    def _(idx1=idx1, ones_x=ones_x):
      run_array(idx1, ones_x, NSUB * nch)

  plsc.subcore_barrier()

  @pl.when(c == 0)
  def _():
    _stage_rows(acc, out0, rb, zb)

  @pl.when(c == 1)
  def _():
    _stage_rows(acc, out1, rb, zb)


_count_pass = pl.kernel(
    _count_body,
    out_type=(jax.ShapeDtypeStruct((NP, FD), f32),
              jax.ShapeDtypeStruct((NP, FD), f32)),
    mesh=_MESH,
    scratch_types=[
        pltpu.VMEM_SHARED((NP, FD), f32),
        pltpu.VMEM((K, FD), f32),
        pltpu.VMEM((STG, FD), f32),
    ] + [pltpu.VMEM((K,), jnp.int32) for _ in range(_NSLOT)]
      + [pltpu.SemaphoreType.DMA for _ in range(2 * _NSLOT)],
    name="sc_counts",
)


BT = 2048
GRID = NP // BT

_row = pl.BlockSpec((BT, FD), lambda i: (i, 0))
_wgt = pl.BlockSpec((FD, FD), lambda i: (0, 0))
_bia = pl.BlockSpec((1, FD), lambda i: (0, 0))
_gat = pl.BlockSpec((1, 1), lambda i: (0, 0))

_CDST, _CNI, _CEI = 0, 1, 2


def _elu(v):
  return jnp.where(v > 0, v, jnp.exp(jnp.minimum(v, 0.0)) - 1.0)


def _dinv_gcn(c0, c1):
  t = c0[...][:, _CDST:_CDST + 1] + c1[...][:, _CDST:_CDST + 1]
  return lax.rsqrt(1.0 + t)


def _inv_pos(c0, c1, lane):
  t = c0[...][:, lane:lane + 1] + c1[...][:, lane:lane + 1]
  return jnp.where(t > 0, 1.0 / t, 0.0)


def _prep1_body(x_r, wg_r, wh_r, ct0_r, ct1_r, y1_r, xwh_r):
  dinv = _dinv_gcn(ct0_r, ct1_r)
  xb = x_r[...]
  y1_r[...] = dinv * jnp.dot(xb, wg_r[...], preferred_element_type=f32)
  xwh_r[...] = jnp.dot(xb, wh_r[...], preferred_element_type=f32)


_prep1 = pl.pallas_call(
    _prep1_body,
    grid=(GRID,),
    in_specs=[_row, _wgt, _wgt, _row, _row],
    out_specs=(_row, _row),
    out_shape=(jax.ShapeDtypeStruct((NP, FD), f32),
               jax.ShapeDtypeStruct((NP, FD), f32)),
    name="tc_prep1",
)


def _mid1_body(accG_r, accH_r, ct0_r, ct1_r, bg1_r, wg2_r,
               y2_r, m1_r):
  dinv = _dinv_gcn(ct0_r, ct1_r)
  xs1 = _elu(dinv * accG_r[...] + bg1_r[...])
  y2_r[...] = dinv * jnp.dot(xs1, wg2_r[...], preferred_element_type=f32)
  m1_r[...] = _inv_pos(ct0_r, ct1_r, _CEI) * accH_r[...]


_mid1 = pl.pallas_call(
    _mid1_body,
    grid=(GRID,),
    in_specs=[_row, _row, _row, _row, _bia, _wgt],
    out_specs=(_row, _row),
    out_shape=(jax.ShapeDtypeStruct((NP, FD), f32),
               jax.ShapeDtypeStruct((NP, FD), f32)),
    name="tc_mid1",
)


def _mid2_body(accG2_r, accH2_r, ct0_r, ct1_r, bg2_r, bh1_r,
               wh2_r, xs_r, yh2_r):
  dinv = _dinv_gcn(ct0_r, ct1_r)
  xs_r[...] = dinv * accG2_r[...] + bg2_r[...]
  xd1 = _elu(_inv_pos(ct0_r, ct1_r, _CNI) * accH2_r[...] + bh1_r[...])
  yh2_r[...] = jnp.dot(xd1, wh2_r[...], preferred_element_type=f32)


_mid2 = pl.pallas_call(
    _mid2_body,
    grid=(GRID,),
    in_specs=[_row, _row, _row, _row, _bia, _bia, _wgt],
    out_specs=(_row, _row),
    out_shape=(jax.ShapeDtypeStruct((NP, FD), f32),
               jax.ShapeDtypeStruct((NP, FD), f32)),
    name="tc_mid2",
)


def _mid3_body(a_r, b_r, ct0_r, ct1_r, m2_r):
  m2_r[...] = _inv_pos(ct0_r, ct1_r, _CEI) * (a_r[...] + b_r[...])


_mid3 = pl.pallas_call(
    _mid3_body,
    grid=(GRID,),
    in_specs=[_row, _row, _row, _row],
    out_specs=_row,
    out_shape=jax.ShapeDtypeStruct((NP, FD), f32),
    name="tc_mid3",
)


def _final_body(a_r, b_r, ct0_r, ct1_r, bh2_r, xs_r, g_r, z_r, xd_r):
  xd = _inv_pos(ct0_r, ct1_r, _CNI) * (a_r[...] + b_r[...]) + bh2_r[...]
  xd_r[...] = xd
  alpha = 1.0 / (1.0 + jnp.exp(-g_r[...]))
  z_r[...] = alpha * xs_r[...] + (1.0 - alpha) * xd


_final = pl.pallas_call(
    _final_body,
    grid=(GRID,),
    in_specs=[_row, _row, _row, _row, _bia, _row, _gat],
    out_specs=(_row, _row),
    out_shape=(jax.ShapeDtypeStruct((NP, FD), f32),
               jax.ShapeDtypeStruct((NP, FD), f32)),
    name="tc_final",
)


def kernel(x, edge_index, hyperedge_index,
           W_g1, b_g1, W_g2, b_g2, W_h1, b_h1, W_h2, b_h2, gate):
  src1 = edge_index[0]
  dst1 = edge_index[1]
  ni1 = hyperedge_index[0]
  ei1 = hyperedge_index[1]

  xp = jnp.pad(x, ((0, NP - NN), (0, 0)))
  zeros128 = jnp.zeros((STG, FD), f32)
  eye = jnp.eye(3, FD, dtype=f32)
  ones_d = jnp.tile(eye[0][None], (K, 1))
  ones_n = jnp.tile(eye[1][None], (K, 1))
  ones_e = jnp.tile(eye[2][None], (K, 1))
  bg1 = b_g1.reshape(1, FD)
  bg2 = b_g2.reshape(1, FD)
  bh1 = b_h1.reshape(1, FD)
  bh2 = b_h2.reshape(1, FD)
  g2 = gate.reshape(1, 1)

  ct0, ct1 = _count_pass(dst1, ni1, ei1, ones_d, ones_n, ones_e, zeros128)

  y1, xwh1 = _prep1(xp, W_g1, W_h1, ct0, ct1)
  accG, accH = _pair_pass(y1, src1, dst1, xwh1, ni1, ei1, zeros128)
  y2, m1 = _mid1(accG, accH, ct0, ct1, bg1, W_g2)
  accG2, accH2 = _pair_pass(y2, src1, dst1, m1, ei1, ni1, zeros128)
  xs, yh2 = _mid2(accG2, accH2, ct0, ct1, bg2, bh1, W_h2)
  a3, b3 = _solo_pass(yh2, ni1, ei1, zeros128)
  m2 = _mid3(a3, b3, ct0, ct1)
  a4, b4 = _solo_pass(m2, ei1, ni1, zeros128)
  zf, xdf = _final(a4, b4, ct0, ct1, bh2, xs, g2)
  return (zf[:NN], xs[:NN], xdf[:NN])
